# Initial kernel scaffold; baseline (speedup 1.0000x reference)
import jax
import jax.numpy as jnp
from jax.experimental import pallas as pl


def kernel(vol, seg_lin, seg_w, seg_ids):
    raise NotImplementedError("write your pallas kernel here")


# speedup vs baseline: 8.5662x; 8.5662x over previous
import functools

import jax
import jax.numpy as jnp
from jax import lax
from jax.experimental import pallas as pl
from jax.experimental.pallas import tpu as pltpu
from jax.experimental.pallas import tpu_sc as plsc

A = 60
U = 64
R = A * U
NC = 2
NS = 16
NW = NC * NS
NRG = 8
ND = NW // NRG
RPW = R // NRG
NBLK = RPW // 16


def _sc_projector(volT, linb, wb, L, Z):
    XY = volT.shape[1]
    DS = Z // ND

    mesh = plsc.VectorSubcoreMesh(core_axis_name="c", subcore_axis_name="s")

    @functools.partial(
        pl.kernel,
        out_type=jax.ShapeDtypeStruct((ND, R, DS), jnp.float32),
        mesh=mesh,
        compiler_params=pltpu.CompilerParams(
            needs_layout_passes=False, use_tc_tiling_on_sc=False),
        scratch_types=[
            pltpu.VMEM((XY, DS), jnp.float32),
            pltpu.VMEM((L, 16), jnp.int32),
            pltpu.VMEM((L, 16), jnp.float32),
            pltpu.VMEM((RPW, DS), jnp.float32),
        ],
    )
    def body(volT_hbm, linb_hbm, wb_hbm, out_hbm, table_v, lin_v, w_v, out_v):
        wid = lax.axis_index("s") * NC + lax.axis_index("c")
        rg = wid // ND
        ds = wid % ND
        pltpu.sync_copy(volT_hbm.at[ds], table_v)
        lane = lax.iota(jnp.int32, 16)

        def block(b, carry):
            gb = rg * NBLK + b
            pltpu.sync_copy(linb_hbm.at[gb], lin_v)
            pltpu.sync_copy(wb_hbm.at[gb], w_v)

            def seg(l, acc):
                lv = lin_v[l]
                wv = w_v[l]
                return tuple(
                    acc[d] + wv * plsc.load_gather(
                        table_v, [lv, jnp.full((16,), d, jnp.int32)])
                    for d in range(DS)
                )

            acc0 = tuple(jnp.zeros((16,), jnp.float32) for _ in range(DS))
            acc = lax.fori_loop(0, L, seg, acc0)
            rows = b * 16 + lane
            for d in range(DS):
                plsc.store_scatter(
                    out_v, [rows, jnp.full((16,), d, jnp.int32)], acc[d])
            return carry

        lax.fori_loop(0, NBLK, block, 0)
        pltpu.sync_copy(out_v, out_hbm.at[ds, pl.ds(rg * RPW, RPW)])

    return body(volT, linb, wb)


def kernel(vol, seg_lin, seg_w, seg_ids):
    B, C, X, Y, Z = vol.shape
    L = seg_lin.size // R
    DS = Z // ND
    volT = vol.reshape(X * Y, ND, DS).transpose(1, 0, 2)
    lin2 = (seg_lin % X) * Y + seg_lin // X
    linb = lin2.reshape(A, U, L).transpose(1, 0, 2) \
               .reshape(R // 16, 16, L).transpose(0, 2, 1)
    wb = seg_w.reshape(A, U, L).transpose(1, 0, 2) \
              .reshape(R // 16, 16, L).transpose(0, 2, 1)
    out = _sc_projector(volT, linb.astype(jnp.int32), wb, L, Z)
    rays = out.transpose(1, 0, 2).reshape(R, Z)
    return rays.reshape(1, 1, U, A, Z)
